# baseline (speedup 1.0000x reference)
import functools

import jax
import jax.numpy as jnp
from jax import lax
from jax.experimental import pallas as pl
from jax.experimental.pallas import tpu as pltpu
from jax.experimental.pallas import tpu_sc as plsc

N_NODES = 100000
N_EDGES = 3200000
D_EDGE = 16

LANES = 128
ROWS_PER_CHUNK = 6
CHUNK_EDGES = LANES * ROWS_PER_CHUNK
NW = 32
NCALLS = 4
CALL_EDGES = N_EDGES // NCALLS
CALL_ROWS = CALL_EDGES // LANES
ROWS_PER_W = CALL_ROWS // NW
EXTRA_W = CALL_ROWS - ROWS_PER_W * NW
FULL_CHUNKS = (ROWS_PER_W // ROWS_PER_CHUNK) // 2 * 2
MAIN_ROWS = FULL_CHUNKS * ROWS_PER_CHUNK
NODES_PER_TILE = N_NODES // 16


def _make_sc_body(ebase, first):
    def body(col_hbm, attr_hbm, *rest):
        if first:
            (out_hbm, acc, idx0, idx1, at0, at1, sem0, sem1) = rest
            prev_hbm = None
        else:
            (prev_hbm, out_hbm, acc, idx0, idx1, at0, at1, sem0,
             sem1) = rest
        i32 = jnp.int32
        c = lax.axis_index("c").astype(i32)
        s = lax.axis_index("s").astype(i32)
        w = c * i32(16) + s

        idxb = (idx0, idx1)
        attb = (at0, at1)
        sems = (sem0, sem1)
        base_n = s * NODES_PER_TILE

        if first:
            z = jnp.zeros((D_EDGE,), jnp.float32)

            def zrow(i, carry):
                at0[i] = z
                return carry

            lax.fori_loop(jnp.int32(0), jnp.int32(CHUNK_EDGES), zrow,
                          jnp.int32(0))

            def zcopy(i, carry):
                pltpu.sync_copy(
                    at0.at[pl.ds(i32(0), CHUNK_EDGES)],
                    acc.at[pl.ds(base_n + i * i32(CHUNK_EDGES),
                                 CHUNK_EDGES)])
                return carry

            nfull = NODES_PER_TILE // CHUNK_EDGES
            lax.fori_loop(jnp.int32(0), jnp.int32(nfull), zcopy,
                          jnp.int32(0))
            rem = NODES_PER_TILE - nfull * CHUNK_EDGES
            if rem:
                pltpu.sync_copy(
                    at0.at[pl.ds(i32(0), rem)],
                    acc.at[pl.ds(base_n + i32(nfull * CHUNK_EDGES), rem)])
        else:
            pltpu.sync_copy(
                prev_hbm.at[c, pl.ds(base_n, NODES_PER_TILE)],
                acc.at[pl.ds(base_n, NODES_PER_TILE)])
        plsc.subcore_barrier()

        base_row = w * i32(ROWS_PER_W) + jnp.minimum(w, i32(EXTRA_W))
        n_rows = i32(ROWS_PER_W) + jnp.where(w < EXTRA_W, i32(1), i32(0))

        def start(kb, b):
            r0 = base_row + kb * i32(ROWS_PER_CHUNK)
            e0 = r0 * i32(LANES)
            for j in range(ROWS_PER_CHUNK):
                pltpu.async_copy(
                    col_hbm.at[i32(1), pl.ds(e0 + i32(j * LANES), LANES)],
                    idxb[b].at[i32(j)], sems[b])
            pltpu.async_copy(attr_hbm.at[pl.ds(e0 + i32(ebase),
                                               CHUNK_EDGES)],
                             attb[b], sems[b])

        def wait(b):
            for j in range(ROWS_PER_CHUNK):
                pltpu.make_async_copy(col_hbm.at[i32(1), pl.ds(0, LANES)],
                                      idxb[b].at[i32(j)], sems[b]).wait()
            pltpu.make_async_copy(attr_hbm.at[pl.ds(0, CHUNK_EDGES)],
                                  attb[b], sems[b]).wait()

        start(0, 0)
        start(1, 1)

        def chunk_pair(i, carry):
            for b in range(2):
                kb = i * jnp.int32(2) + jnp.int32(b)
                wait(b)
                for j in range(ROWS_PER_CHUNK):
                    pltpu.sync_copy(
                        attb[b].at[pl.ds(i32(j * LANES), LANES)],
                        acc.at[idxb[b].at[i32(j)]], add=True)
                pl.when(kb + 2 < FULL_CHUNKS)(lambda: start(kb + 2, b))
            return carry

        lax.fori_loop(jnp.int32(0), jnp.int32(FULL_CHUNKS // 2), chunk_pair,
                      jnp.int32(0))

        def tail_row(r, carry):
            rr = base_row + r
            e0 = rr * jnp.int32(LANES)
            pltpu.async_copy(col_hbm.at[jnp.int32(1), pl.ds(e0, LANES)],
                             idx0.at[jnp.int32(0)], sem0)
            pltpu.async_copy(attr_hbm.at[pl.ds(e0 + i32(ebase), LANES)],
                             at0.at[pl.ds(jnp.int32(0), LANES)], sem0)
            pltpu.make_async_copy(col_hbm.at[jnp.int32(1), pl.ds(0, LANES)],
                                  idx0.at[jnp.int32(0)], sem0).wait()
            pltpu.make_async_copy(attr_hbm.at[pl.ds(0, LANES)],
                                  at0.at[pl.ds(jnp.int32(0), LANES)],
                                  sem0).wait()
            pltpu.sync_copy(at0.at[pl.ds(jnp.int32(0), LANES)],
                            acc.at[idx0.at[jnp.int32(0)]], add=True)
            return carry

        lax.fori_loop(jnp.int32(MAIN_ROWS), n_rows, tail_row, jnp.int32(0))

        plsc.subcore_barrier()

        pltpu.sync_copy(acc.at[pl.ds(base_n, NODES_PER_TILE)],
                        out_hbm.at[c, pl.ds(base_n, NODES_PER_TILE)])

    return body


_MESH = plsc.VectorSubcoreMesh(core_axis_name="c", subcore_axis_name="s",
                               num_cores=2, num_subcores=16)
_PARAMS = pltpu.CompilerParams(use_tc_tiling_on_sc=False,
                               needs_layout_passes=False)
_SCRATCH = [
    pltpu.VMEM_SHARED((N_NODES, D_EDGE), jnp.float32),
    pltpu.VMEM((ROWS_PER_CHUNK, LANES), jnp.int32),
    pltpu.VMEM((ROWS_PER_CHUNK, LANES), jnp.int32),
    pltpu.VMEM((CHUNK_EDGES, D_EDGE), jnp.float32),
    pltpu.VMEM((CHUNK_EDGES, D_EDGE), jnp.float32),
    pltpu.SemaphoreType.DMA,
    pltpu.SemaphoreType.DMA,
]

_sc_calls = [
    pl.kernel(
        _make_sc_body(k * CALL_EDGES, k == 0),
        out_type=jax.ShapeDtypeStruct((2, N_NODES, D_EDGE), jnp.float32),
        mesh=_MESH,
        compiler_params=_PARAMS,
        scratch_types=_SCRATCH,
    )
    for k in range(NCALLS)
]


CMB_ROWS = N_NODES // NW
CMB_CHUNK = 625


def _cmb_body(p_hbm, out_hbm, a_v, b_v, o_v, sem):
    i32 = jnp.int32
    c = lax.axis_index("c").astype(i32)
    s = lax.axis_index("s").astype(i32)
    w = c * i32(16) + s
    base = w * i32(CMB_ROWS)

    def chunk(k, carry):
        r0 = base + k * i32(CMB_CHUNK)
        pltpu.async_copy(p_hbm.at[i32(0), pl.ds(r0, CMB_CHUNK)], a_v, sem)
        pltpu.async_copy(p_hbm.at[i32(1), pl.ds(r0, CMB_CHUNK)], b_v, sem)
        pltpu.make_async_copy(p_hbm.at[i32(0), pl.ds(0, CMB_CHUNK)], a_v,
                              sem).wait()
        pltpu.make_async_copy(p_hbm.at[i32(1), pl.ds(0, CMB_CHUNK)], b_v,
                              sem).wait()

        def row(i, cc):
            o_v[i] = a_v[i] + b_v[i]
            return cc

        lax.fori_loop(jnp.int32(0), jnp.int32(CMB_CHUNK), row, jnp.int32(0))
        pltpu.sync_copy(o_v, out_hbm.at[pl.ds(r0, CMB_CHUNK)])
        return carry

    lax.fori_loop(jnp.int32(0), jnp.int32(CMB_ROWS // CMB_CHUNK), chunk,
                  jnp.int32(0))


_sc_combine = functools.partial(
    pl.kernel,
    out_type=jax.ShapeDtypeStruct((N_NODES, D_EDGE), jnp.float32),
    mesh=_MESH,
    compiler_params=_PARAMS,
    scratch_types=[
        pltpu.VMEM((CMB_CHUNK, D_EDGE), jnp.float32),
        pltpu.VMEM((CMB_CHUNK, D_EDGE), jnp.float32),
        pltpu.VMEM((CMB_CHUNK, D_EDGE), jnp.float32),
        pltpu.SemaphoreType.DMA,
    ],
)(_cmb_body)


@jax.jit
def kernel(x, edge_index, edge_attr, u, batch):
    cols = [
        edge_index[:, k * CALL_EDGES:(k + 1) * CALL_EDGES].astype(jnp.int32)
        for k in range(NCALLS)
    ]
    p = _sc_calls[0](cols[0], edge_attr)
    for k in range(1, NCALLS):
        p = _sc_calls[k](cols[k], edge_attr, p)
    return _sc_combine(p)


# speedup vs baseline: 1.0386x; 1.0386x over previous
import functools

import jax
import jax.numpy as jnp
from jax import lax
from jax.experimental import pallas as pl
from jax.experimental.pallas import tpu as pltpu
from jax.experimental.pallas import tpu_sc as plsc

N_NODES = 100000
N_EDGES = 3200000
D_EDGE = 16

LANES = 128
ROWS_PER_CHUNK = 6
CHUNK_EDGES = LANES * ROWS_PER_CHUNK
N_ROWS = N_EDGES // LANES
NW = 32
ROWS_PER_W = N_ROWS // NW
EXTRA_W = N_ROWS - ROWS_PER_W * NW
FULL_CHUNKS = (ROWS_PER_W // ROWS_PER_CHUNK) // 2 * 2
MAIN_ROWS = FULL_CHUNKS * ROWS_PER_CHUNK
NODES_PER_TILE = N_NODES // 16


def _sc_body(col_hbm, attr_hbm, out_hbm,
             acc, idx0, idx1, at0, at1, sem0, sem1, ssc0, ssc1):
    i32 = jnp.int32
    c = lax.axis_index("c").astype(i32)
    s = lax.axis_index("s").astype(i32)
    w = c * i32(16) + s

    idxb = (idx0, idx1)
    attb = (at0, at1)
    sems = (sem0, sem1)
    sscs = (ssc0, ssc1)

    z = jnp.zeros((D_EDGE,), jnp.float32)

    def zrow(i, carry):
        at0[i] = z
        return carry

    lax.fori_loop(jnp.int32(0), jnp.int32(CHUNK_EDGES), zrow, jnp.int32(0))
    base_n = s * NODES_PER_TILE

    def zcopy(i, carry):
        pltpu.sync_copy(at0.at[pl.ds(i32(0), CHUNK_EDGES)],
                        acc.at[pl.ds(base_n + i * i32(CHUNK_EDGES),
                                     CHUNK_EDGES)])
        return carry

    nfull = NODES_PER_TILE // CHUNK_EDGES
    lax.fori_loop(jnp.int32(0), jnp.int32(nfull), zcopy, jnp.int32(0))
    rem = NODES_PER_TILE - nfull * CHUNK_EDGES
    if rem:
        pltpu.sync_copy(
            at0.at[pl.ds(i32(0), rem)],
            acc.at[pl.ds(base_n + i32(nfull * CHUNK_EDGES), rem)])
    plsc.subcore_barrier()

    base_row = w * i32(ROWS_PER_W) + jnp.minimum(w, i32(EXTRA_W))
    n_rows = i32(ROWS_PER_W) + jnp.where(w < EXTRA_W, i32(1), i32(0))

    def start(kb, b):
        r0 = base_row + kb * i32(ROWS_PER_CHUNK)
        e0 = r0 * i32(LANES)
        for j in range(ROWS_PER_CHUNK):
            pltpu.async_copy(
                col_hbm.at[i32(1), pl.ds(e0 + i32(j * LANES), LANES)],
                idxb[b].at[i32(j)], sems[b])
        pltpu.async_copy(attr_hbm.at[pl.ds(e0, CHUNK_EDGES)],
                         attb[b], sems[b])

    def wait(b):
        for j in range(ROWS_PER_CHUNK):
            pltpu.make_async_copy(col_hbm.at[i32(1), pl.ds(0, LANES)],
                                  idxb[b].at[i32(j)], sems[b]).wait()
        pltpu.make_async_copy(attr_hbm.at[pl.ds(0, CHUNK_EDGES)], attb[b],
                              sems[b]).wait()

    def fire(b):
        for j in range(ROWS_PER_CHUNK):
            pltpu.async_copy(attb[b].at[pl.ds(i32(j * LANES), LANES)],
                             acc.at[idxb[b].at[i32(j)]], sscs[b], add=True)

    def drain(b):
        for j in range(ROWS_PER_CHUNK):
            pltpu.make_async_copy(attb[b].at[pl.ds(i32(j * LANES), LANES)],
                                  acc.at[idxb[b].at[i32(j)]],
                                  sscs[b]).wait()

    start(0, 0)

    def chunk_pair(i, carry):
        for b in range(2):
            kb = i * jnp.int32(2) + jnp.int32(b)
            wait(b)
            fire(b)
            pl.when(kb > 0)(lambda: drain(1 - b))
            pl.when(kb + 1 < FULL_CHUNKS)(lambda: start(kb + 1, 1 - b))
        return carry

    lax.fori_loop(jnp.int32(0), jnp.int32(FULL_CHUNKS // 2), chunk_pair,
                  jnp.int32(0))
    drain((FULL_CHUNKS - 1) % 2)

    def tail_row(r, carry):
        rr = base_row + r
        pltpu.async_copy(
            col_hbm.at[jnp.int32(1), pl.ds(rr * jnp.int32(LANES), LANES)],
            idx0.at[jnp.int32(0)], sem0)
        pltpu.async_copy(attr_hbm.at[pl.ds(rr * jnp.int32(LANES), LANES)],
                         at0.at[pl.ds(jnp.int32(0), LANES)], sem0)
        pltpu.make_async_copy(col_hbm.at[jnp.int32(1), pl.ds(0, LANES)],
                              idx0.at[jnp.int32(0)], sem0).wait()
        pltpu.make_async_copy(attr_hbm.at[pl.ds(0, LANES)],
                              at0.at[pl.ds(jnp.int32(0), LANES)],
                              sem0).wait()
        pltpu.sync_copy(at0.at[pl.ds(jnp.int32(0), LANES)],
                        acc.at[idx0.at[jnp.int32(0)]], add=True)
        return carry

    lax.fori_loop(jnp.int32(MAIN_ROWS), n_rows, tail_row, jnp.int32(0))

    plsc.subcore_barrier()

    pltpu.sync_copy(acc.at[pl.ds(base_n, NODES_PER_TILE)],
                    out_hbm.at[c, pl.ds(base_n, NODES_PER_TILE)])


_sc_scatter = functools.partial(
    pl.kernel,
    out_type=jax.ShapeDtypeStruct((2, N_NODES, D_EDGE), jnp.float32),
    mesh=plsc.VectorSubcoreMesh(core_axis_name="c", subcore_axis_name="s",
                                num_cores=2, num_subcores=16),
    compiler_params=pltpu.CompilerParams(use_tc_tiling_on_sc=False,
                                         needs_layout_passes=False),
    scratch_types=[
        pltpu.VMEM_SHARED((N_NODES, D_EDGE), jnp.float32),
        pltpu.VMEM((ROWS_PER_CHUNK, LANES), jnp.int32),
        pltpu.VMEM((ROWS_PER_CHUNK, LANES), jnp.int32),
        pltpu.VMEM((CHUNK_EDGES, D_EDGE), jnp.float32),
        pltpu.VMEM((CHUNK_EDGES, D_EDGE), jnp.float32),
        pltpu.SemaphoreType.DMA,
        pltpu.SemaphoreType.DMA,
        pltpu.SemaphoreType.DMA,
        pltpu.SemaphoreType.DMA,
    ],
)(_sc_body)


CMB_ROWS = N_NODES // NW
CMB_CHUNK = 625


def _cmb_body(p_hbm, out_hbm, a_v, b_v, o_v, sem):
    i32 = jnp.int32
    c = lax.axis_index("c").astype(i32)
    s = lax.axis_index("s").astype(i32)
    w = c * i32(16) + s
    base = w * i32(CMB_ROWS)

    def chunk(k, carry):
        r0 = base + k * i32(CMB_CHUNK)
        pltpu.async_copy(p_hbm.at[i32(0), pl.ds(r0, CMB_CHUNK)], a_v, sem)
        pltpu.async_copy(p_hbm.at[i32(1), pl.ds(r0, CMB_CHUNK)], b_v, sem)
        pltpu.make_async_copy(p_hbm.at[i32(0), pl.ds(0, CMB_CHUNK)], a_v,
                              sem).wait()
        pltpu.make_async_copy(p_hbm.at[i32(1), pl.ds(0, CMB_CHUNK)], b_v,
                              sem).wait()

        def row(i, cc):
            o_v[i] = a_v[i] + b_v[i]
            return cc

        lax.fori_loop(jnp.int32(0), jnp.int32(CMB_CHUNK), row, jnp.int32(0))
        pltpu.sync_copy(o_v, out_hbm.at[pl.ds(r0, CMB_CHUNK)])
        return carry

    lax.fori_loop(jnp.int32(0), jnp.int32(CMB_ROWS // CMB_CHUNK), chunk,
                  jnp.int32(0))


_sc_combine = functools.partial(
    pl.kernel,
    out_type=jax.ShapeDtypeStruct((N_NODES, D_EDGE), jnp.float32),
    mesh=plsc.VectorSubcoreMesh(core_axis_name="c", subcore_axis_name="s",
                                num_cores=2, num_subcores=16),
    compiler_params=pltpu.CompilerParams(use_tc_tiling_on_sc=False,
                                         needs_layout_passes=False),
    scratch_types=[
        pltpu.VMEM((CMB_CHUNK, D_EDGE), jnp.float32),
        pltpu.VMEM((CMB_CHUNK, D_EDGE), jnp.float32),
        pltpu.VMEM((CMB_CHUNK, D_EDGE), jnp.float32),
        pltpu.SemaphoreType.DMA,
    ],
)(_cmb_body)


def _cast_body(ei_ref, o_ref):
    o_ref[...] = ei_ref[1:2, :].astype(jnp.int32)


def _cast_col(edge_index):
    out = pl.pallas_call(
        _cast_body,
        grid=(25,),
        in_specs=[pl.BlockSpec((2, 128000), lambda i: (jnp.int32(0), i))],
        out_specs=pl.BlockSpec((1, 128000), lambda i: (jnp.int32(0), i)),
        out_shape=jax.ShapeDtypeStruct((1, 25 * 128000), jnp.int32),
    )(edge_index)
    return out.reshape(N_EDGES)


def _combine(p):
    return _sc_combine(p)


@jax.jit
def kernel(x, edge_index, edge_attr, u, batch):
    col = edge_index.astype(jnp.int32)
    partials = _sc_scatter(col, edge_attr)
    return _combine(partials)


# speedup vs baseline: 1.0539x; 1.0147x over previous
import functools

import jax
import jax.numpy as jnp
from jax import lax
from jax.experimental import pallas as pl
from jax.experimental.pallas import tpu as pltpu
from jax.experimental.pallas import tpu_sc as plsc

N_NODES = 100000
N_EDGES = 3200000
D_EDGE = 16

LANES = 128
ROWS_PER_CHUNK = 6
CHUNK_EDGES = LANES * ROWS_PER_CHUNK
N_ROWS = N_EDGES // LANES
NW = 32
ROWS_PER_W = N_ROWS // NW
EXTRA_W = N_ROWS - ROWS_PER_W * NW
FULL_CHUNKS = (ROWS_PER_W // ROWS_PER_CHUNK) // 2 * 2
MAIN_ROWS = FULL_CHUNKS * ROWS_PER_CHUNK
NODES_PER_TILE = N_NODES // 16


def _sc_body(col_hbm, attr_hbm, out_hbm,
             acc, idx0, idx1, at0, at1, sem0, sem1):
    i32 = jnp.int32
    c = lax.axis_index("c").astype(i32)
    s = lax.axis_index("s").astype(i32)
    w = c * i32(16) + s

    idxb = (idx0, idx1)
    attb = (at0, at1)
    sems = (sem0, sem1)

    z = jnp.zeros((D_EDGE,), jnp.float32)

    def zrow(i, carry):
        at0[i] = z
        return carry

    lax.fori_loop(jnp.int32(0), jnp.int32(CHUNK_EDGES), zrow, jnp.int32(0))
    base_n = s * NODES_PER_TILE

    def zcopy(i, carry):
        pltpu.sync_copy(at0.at[pl.ds(i32(0), CHUNK_EDGES)],
                        acc.at[pl.ds(base_n + i * i32(CHUNK_EDGES),
                                     CHUNK_EDGES)])
        return carry

    nfull = NODES_PER_TILE // CHUNK_EDGES
    lax.fori_loop(jnp.int32(0), jnp.int32(nfull), zcopy, jnp.int32(0))
    rem = NODES_PER_TILE - nfull * CHUNK_EDGES
    if rem:
        pltpu.sync_copy(
            at0.at[pl.ds(i32(0), rem)],
            acc.at[pl.ds(base_n + i32(nfull * CHUNK_EDGES), rem)])
    plsc.subcore_barrier()

    base_row = w * i32(ROWS_PER_W) + jnp.minimum(w, i32(EXTRA_W))
    n_rows = i32(ROWS_PER_W) + jnp.where(w < EXTRA_W, i32(1), i32(0))

    def start(kb, b):
        r0 = base_row + kb * i32(ROWS_PER_CHUNK)
        e0 = r0 * i32(LANES)
        for j in range(ROWS_PER_CHUNK):
            pltpu.async_copy(
                col_hbm.at[i32(1), pl.ds(e0 + i32(j * LANES), LANES)],
                idxb[b].at[i32(j)], sems[b])
        pltpu.async_copy(attr_hbm.at[pl.ds(e0, CHUNK_EDGES)],
                         attb[b], sems[b])

    def wait(b):
        for j in range(ROWS_PER_CHUNK):
            pltpu.make_async_copy(col_hbm.at[i32(1), pl.ds(0, LANES)],
                                  idxb[b].at[i32(j)], sems[b]).wait()
        pltpu.make_async_copy(attr_hbm.at[pl.ds(0, CHUNK_EDGES)], attb[b],
                              sems[b]).wait()

    start(0, 0)
    start(1, 1)

    def chunk_pair(i, carry):
        for b in range(2):
            kb = i * jnp.int32(2) + jnp.int32(b)
            wait(b)
            for j in range(ROWS_PER_CHUNK):
                pltpu.sync_copy(attb[b].at[pl.ds(i32(j * LANES), LANES)],
                                acc.at[idxb[b].at[i32(j)]], add=True)
            pl.when(kb + 2 < FULL_CHUNKS)(lambda: start(kb + 2, b))
        return carry

    lax.fori_loop(jnp.int32(0), jnp.int32(FULL_CHUNKS // 2), chunk_pair,
                  jnp.int32(0))

    def tail_row(r, carry):
        rr = base_row + r
        pltpu.async_copy(
            col_hbm.at[jnp.int32(1), pl.ds(rr * jnp.int32(LANES), LANES)],
            idx0.at[jnp.int32(0)], sem0)
        pltpu.async_copy(attr_hbm.at[pl.ds(rr * jnp.int32(LANES), LANES)],
                         at0.at[pl.ds(jnp.int32(0), LANES)], sem0)
        pltpu.make_async_copy(col_hbm.at[jnp.int32(1), pl.ds(0, LANES)],
                              idx0.at[jnp.int32(0)], sem0).wait()
        pltpu.make_async_copy(attr_hbm.at[pl.ds(0, LANES)],
                              at0.at[pl.ds(jnp.int32(0), LANES)],
                              sem0).wait()
        pltpu.sync_copy(at0.at[pl.ds(jnp.int32(0), LANES)],
                        acc.at[idx0.at[jnp.int32(0)]], add=True)
        return carry

    lax.fori_loop(jnp.int32(MAIN_ROWS), n_rows, tail_row, jnp.int32(0))

    plsc.subcore_barrier()

    pltpu.sync_copy(acc.at[pl.ds(base_n, NODES_PER_TILE)],
                    out_hbm.at[c, pl.ds(base_n, NODES_PER_TILE)])


_sc_scatter = functools.partial(
    pl.kernel,
    out_type=jax.ShapeDtypeStruct((2, N_NODES, D_EDGE), jnp.float32),
    mesh=plsc.VectorSubcoreMesh(core_axis_name="c", subcore_axis_name="s",
                                num_cores=2, num_subcores=16),
    compiler_params=pltpu.CompilerParams(use_tc_tiling_on_sc=False,
                                         needs_layout_passes=False),
    scratch_types=[
        pltpu.VMEM_SHARED((N_NODES, D_EDGE), jnp.float32),
        pltpu.VMEM((ROWS_PER_CHUNK, LANES), jnp.int32),
        pltpu.VMEM((ROWS_PER_CHUNK, LANES), jnp.int32),
        pltpu.VMEM((CHUNK_EDGES, D_EDGE), jnp.float32),
        pltpu.VMEM((CHUNK_EDGES, D_EDGE), jnp.float32),
        pltpu.SemaphoreType.DMA,
        pltpu.SemaphoreType.DMA,
    ],
)(_sc_body)


CMB_ROWS = N_NODES // NW
CMB_CHUNK = 625


def _cmb_body(p_hbm, out_hbm, a_v, b_v, o_v, sem):
    i32 = jnp.int32
    c = lax.axis_index("c").astype(i32)
    s = lax.axis_index("s").astype(i32)
    w = c * i32(16) + s
    base = w * i32(CMB_ROWS)

    def chunk(k, carry):
        r0 = base + k * i32(CMB_CHUNK)
        pltpu.async_copy(p_hbm.at[i32(0), pl.ds(r0, CMB_CHUNK)], a_v, sem)
        pltpu.async_copy(p_hbm.at[i32(1), pl.ds(r0, CMB_CHUNK)], b_v, sem)
        pltpu.make_async_copy(p_hbm.at[i32(0), pl.ds(0, CMB_CHUNK)], a_v,
                              sem).wait()
        pltpu.make_async_copy(p_hbm.at[i32(1), pl.ds(0, CMB_CHUNK)], b_v,
                              sem).wait()

        def row(i, cc):
            o_v[i] = a_v[i] + b_v[i]
            return cc

        lax.fori_loop(jnp.int32(0), jnp.int32(CMB_CHUNK), row, jnp.int32(0))
        pltpu.sync_copy(o_v, out_hbm.at[pl.ds(r0, CMB_CHUNK)])
        return carry

    lax.fori_loop(jnp.int32(0), jnp.int32(CMB_ROWS // CMB_CHUNK), chunk,
                  jnp.int32(0))


_sc_combine = functools.partial(
    pl.kernel,
    out_type=jax.ShapeDtypeStruct((N_NODES, D_EDGE), jnp.float32),
    mesh=plsc.VectorSubcoreMesh(core_axis_name="c", subcore_axis_name="s",
                                num_cores=2, num_subcores=16),
    compiler_params=pltpu.CompilerParams(use_tc_tiling_on_sc=False,
                                         needs_layout_passes=False),
    scratch_types=[
        pltpu.VMEM((CMB_CHUNK, D_EDGE), jnp.float32),
        pltpu.VMEM((CMB_CHUNK, D_EDGE), jnp.float32),
        pltpu.VMEM((CMB_CHUNK, D_EDGE), jnp.float32),
        pltpu.SemaphoreType.DMA,
    ],
)(_cmb_body)


def _cast_body(ei_ref, o_ref):
    o_ref[...] = ei_ref[1:2, :].astype(jnp.int32)


def _cast_col(edge_index):
    out = pl.pallas_call(
        _cast_body,
        grid=(25,),
        in_specs=[pl.BlockSpec((2, 128000), lambda i: (jnp.int32(0), i))],
        out_specs=pl.BlockSpec((1, 128000), lambda i: (jnp.int32(0), i)),
        out_shape=jax.ShapeDtypeStruct((1, 25 * 128000), jnp.int32),
    )(edge_index)
    return out.reshape(N_EDGES)


def _combine(p):
    return _sc_combine(p)


@jax.jit
def kernel(x, edge_index, edge_attr, u, batch):
    col = edge_index.astype(jnp.int32)
    partials = _sc_scatter(col, edge_attr)
    return _combine(partials)
